# baseline (speedup 1.0000x reference)
import functools

import numpy as np
import jax
import jax.numpy as jnp
from jax import lax
from jax.experimental import pallas as pl
from jax.experimental.pallas import tpu as pltpu
from jax.experimental.pallas import tpu_sc as plsc

NUM_G = 16
L = 16
NC = 2
NS = 16
NW = NC * NS
B = 1024
TC_PER_B = B // 128

_MAGIC = np.int32(0x5F3759DF)


def _rsqrt_newton(s2):
    i = lax.bitcast_convert_type(s2, jnp.int32)
    y = lax.bitcast_convert_type(_MAGIC - (i >> 1), jnp.float32)
    for _ in range(2):
        y = y * (np.float32(1.5) - np.float32(0.5) * s2 * y * y)
    return y


def _compute_block(rows, obuf, par, ngroups):
    eidx0 = lax.iota(jnp.int32, L)
    c0 = jnp.zeros((L,), jnp.int32)
    c1 = jnp.ones((L,), jnp.int32)
    c2 = jnp.full((L,), 2, jnp.int32)
    shift_v = par[0, :]
    neg_inv = par[1, :]
    sg = [shift_v[g] for g in range(NUM_G)]
    ng = [neg_inv[g] for g in range(NUM_G)]

    def grp(k, carry):
        e0 = k * L
        eidx = e0 + eidx0
        xj = plsc.load_gather(rows, [eidx, c0])
        yj = plsc.load_gather(rows, [eidx, c1])
        zj = plsc.load_gather(rows, [eidx, c2])
        eidx2 = eidx + B
        xi = plsc.load_gather(rows, [eidx2, c0])
        yi = plsc.load_gather(rows, [eidx2, c1])
        zi = plsc.load_gather(rows, [eidx2, c2])
        dx = xj - xi
        dy = yj - yi
        dz = zj - zi
        s2 = dx * dx + dy * dy + dz * dz
        n = s2 * _rsqrt_newton(s2)
        n = jnp.where(s2 > np.float32(0.0), n, np.float32(0.0))
        cl = k // (128 // L)
        l0 = (k % (128 // L)) * L
        for g in range(NUM_G):
            t = n - sg[g]
            obuf[g // 8, cl, g % 8, pl.ds(l0, L)] = jnp.exp(t * t * ng[g])
        return carry

    lax.fori_loop(0, ngroups, grp, 0, unroll=4)


def _body(npad, pairs_total, rem_tc, pos8, ei, shift, scale, out,
          shared, idx0, idx1, rows0, rows1, obuf0, obuf1, par,
          sem_s, sem_x0, sem_x1, sem_g0, sem_g1, sem_o0, sem_o1):
    sid = lax.axis_index("s")
    wid = sid * NC + lax.axis_index("c")

    rows_per_tile = npad // NS
    pltpu.sync_copy(pos8.at[pl.ds(sid * rows_per_tile, rows_per_tile)],
                    shared.at[pl.ds(sid * rows_per_tile, rows_per_tile)])
    pltpu.sync_copy(shift, par.at[0])
    pltpu.sync_copy(scale, par.at[1])
    sc = par[1, :]
    par[1, :] = np.float32(-0.5) / (sc * sc)
    plsc.subcore_barrier()

    ppw = pairs_total // NW
    extra = pairs_total % NW
    npair = ppw + jnp.where(wid < extra, 1, 0)
    pair_base = ppw * wid + jnp.minimum(wid, extra)
    nb = 2 * npair

    idx = (idx0, idx1)
    rows = (rows0, rows1)
    obuf = (obuf0, obuf1)
    sem_x = (sem_x0, sem_x1)
    sem_g = (sem_g0, sem_g1)
    sem_o = (sem_o0, sem_o1)

    def block_off(b):
        return (pair_base * 2 + b) * B

    def issue_idx(b, p):
        off = block_off(b)
        pltpu.async_copy(ei.at[0, pl.ds(off, B)], idx[p].at[pl.ds(0, B)],
                         sem_x[p])
        pltpu.async_copy(ei.at[1, pl.ds(off, B)], idx[p].at[pl.ds(B, B)],
                         sem_x[p])

    def wait_idx(b, p):
        pltpu.make_async_copy(ei.at[0, pl.ds(0, B)],
                              idx[p].at[pl.ds(0, B)], sem_x[p]).wait()
        pltpu.make_async_copy(ei.at[1, pl.ds(0, B)],
                              idx[p].at[pl.ds(B, B)], sem_x[p]).wait()

    def issue_gather(p):
        pltpu.async_copy(shared.at[idx[p]], rows[p], sem_g[p])

    def wait_gather(p):
        pltpu.make_async_copy(shared.at[idx[p]], rows[p], sem_g[p]).wait()

    def out_slice(b):
        return out.at[:, pl.ds((pair_base * 2 + b) * TC_PER_B, TC_PER_B)]

    def issue_out(b, p):
        pltpu.async_copy(obuf[p], out_slice(b), sem_o[p])

    def wait_out(p):
        pltpu.make_async_copy(obuf[p], out_slice(0), sem_o[p]).wait()

    @pl.when(nb >= 1)
    def _():
        issue_idx(0, 0)

    @pl.when(nb >= 2)
    def _():
        issue_idx(1, 1)

    @pl.when(nb >= 1)
    def _():
        wait_idx(0, 0)
        issue_gather(0)

    def pair(p, carry):
        for half in (0, 1):
            b = 2 * p + half
            q = 1 - half
            wait_gather(half)

            @pl.when(b + 2 < nb)
            def _():
                issue_idx(b + 2, half)

            @pl.when(b + 1 < nb)
            def _():
                wait_idx(b + 1, q)
                issue_gather(q)

            @pl.when(b >= 2)
            def _():
                wait_out(half)

            _compute_block(rows[half], obuf[half], par, B // L)
            issue_out(b, half)
        return carry

    lax.fori_loop(0, npair, pair, 0, unroll=False)

    @pl.when(nb >= 2)
    def _():
        wait_out(0)

    @pl.when(nb >= 1)
    def _():
        wait_out(1)

    @pl.when(wid < rem_tc)
    def _():
        tc0 = pairs_total * 2 * TC_PER_B + wid
        off = tc0 * 128

        def zero(k, carry):
            idx0[pl.ds(k * L, L)] = jnp.zeros((L,), jnp.int32)
            return carry

        lax.fori_loop(0, 2 * B // L, zero, 0, unroll=False)
        pltpu.sync_copy(ei.at[0, pl.ds(off, 128)], idx0.at[pl.ds(0, 128)])
        pltpu.sync_copy(ei.at[1, pl.ds(off, 128)], idx0.at[pl.ds(B, 128)])
        pltpu.async_copy(shared.at[idx0], rows0, sem_g0).wait()
        _compute_block(rows0, obuf0, par, 128 // L)
        pltpu.sync_copy(obuf0.at[:, pl.ds(0, 1)], out.at[:, pl.ds(tc0, 1)])


def kernel(pos, edge_index, shift, scale):
    n_nodes = pos.shape[0]
    n_edges = edge_index.shape[1]
    ei = edge_index.astype(jnp.int32)
    npad = -(-n_nodes // NS) * NS
    pos8 = jnp.pad(pos.astype(jnp.float32),
                   ((0, npad - n_nodes), (0, 8 - pos.shape[1])))

    e_pad = -(-n_edges // 128) * 128
    if e_pad != n_edges:
        ei = jnp.pad(ei, ((0, 0), (0, e_pad - n_edges)))
    ec = e_pad // 128
    pairs_total = ec // (2 * TC_PER_B)
    rem_tc = ec % (2 * TC_PER_B)
    assert rem_tc < NW

    mesh = plsc.VectorSubcoreMesh(core_axis_name="c", subcore_axis_name="s")
    f = pl.kernel(
        functools.partial(_body, npad, pairs_total, rem_tc),
        out_type=jax.ShapeDtypeStruct((2, ec, 8, 128), jnp.float32),
        mesh=mesh,
        scratch_types=[
            pltpu.VMEM_SHARED((npad, 8), jnp.float32),
            pltpu.VMEM((2 * B,), jnp.int32),
            pltpu.VMEM((2 * B,), jnp.int32),
            pltpu.VMEM((2 * B, 8), jnp.float32),
            pltpu.VMEM((2 * B, 8), jnp.float32),
            pltpu.VMEM((2, TC_PER_B, 8, 128), jnp.float32),
            pltpu.VMEM((2, TC_PER_B, 8, 128), jnp.float32),
            pltpu.VMEM((2, NUM_G), jnp.float32),
            pltpu.SemaphoreType.DMA,
            pltpu.SemaphoreType.DMA,
            pltpu.SemaphoreType.DMA,
            pltpu.SemaphoreType.DMA,
            pltpu.SemaphoreType.DMA,
            pltpu.SemaphoreType.DMA,
            pltpu.SemaphoreType.DMA,
        ],
        compiler_params=pltpu.CompilerParams(
            needs_layout_passes=False,
            use_tc_tiling_on_sc=False,
        ),
        name="gaussian_edge_embed_sc",
    )
    out = f(pos8, ei, shift.astype(jnp.float32), scale.astype(jnp.float32))
    res = out.transpose(1, 3, 0, 2).reshape(e_pad, NUM_G)
    if e_pad != n_edges:
        res = res[:n_edges]
    return res


# speedup vs baseline: 1.0022x; 1.0022x over previous
import functools

import numpy as np
import jax
import jax.numpy as jnp
from jax import lax
from jax.experimental import pallas as pl
from jax.experimental.pallas import tpu as pltpu
from jax.experimental.pallas import tpu_sc as plsc

NUM_G = 16
L = 16
NC = 2
NS = 16
NW = NC * NS
B = 1024
TC_PER_B = B // 128

_MAGIC = np.int32(0x5F3759DF)


def _rsqrt_newton(s2):
    i = lax.bitcast_convert_type(s2, jnp.int32)
    y = lax.bitcast_convert_type(_MAGIC - (i >> 1), jnp.float32)
    for _ in range(2):
        y = y * (np.float32(1.5) - np.float32(0.5) * s2 * y * y)
    return y


def _compute_block(rows, obuf, par, ngroups):
    eidx0 = lax.iota(jnp.int32, L)
    c0 = jnp.zeros((L,), jnp.int32)
    c1 = jnp.ones((L,), jnp.int32)
    c2 = jnp.full((L,), 2, jnp.int32)
    shift_v = par[0, :]
    neg_inv = par[1, :]
    sg = [shift_v[g] for g in range(NUM_G)]
    ng = [neg_inv[g] for g in range(NUM_G)]

    def grp(k, carry):
        e0 = k * L
        eidx = e0 + eidx0
        xj = plsc.load_gather(rows, [eidx, c0])
        yj = plsc.load_gather(rows, [eidx, c1])
        zj = plsc.load_gather(rows, [eidx, c2])
        eidx2 = eidx + B
        xi = plsc.load_gather(rows, [eidx2, c0])
        yi = plsc.load_gather(rows, [eidx2, c1])
        zi = plsc.load_gather(rows, [eidx2, c2])
        dx = xj - xi
        dy = yj - yi
        dz = zj - zi
        s2 = dx * dx + dy * dy + dz * dz
        n = s2 * _rsqrt_newton(s2)
        n = jnp.where(s2 > np.float32(0.0), n, np.float32(0.0))
        cl = k // (128 // L)
        l0 = (k % (128 // L)) * L
        for g in range(NUM_G):
            t = n - sg[g]
            obuf[g // 8, cl, g % 8, pl.ds(l0, L)] = jnp.exp(t * t * ng[g])
        return carry

    lax.fori_loop(0, ngroups, grp, 0, unroll=8)


def _body(npad, pairs_total, rem_tc, pos8, ei, shift, scale, out,
          shared, idx0, idx1, rows0, rows1, obuf0, obuf1, par,
          sem_s, sem_x0, sem_x1, sem_g0, sem_g1, sem_o0, sem_o1):
    sid = lax.axis_index("s")
    wid = sid * NC + lax.axis_index("c")

    rows_per_tile = npad // NS
    pltpu.sync_copy(pos8.at[pl.ds(sid * rows_per_tile, rows_per_tile)],
                    shared.at[pl.ds(sid * rows_per_tile, rows_per_tile)])
    pltpu.sync_copy(shift, par.at[0])
    pltpu.sync_copy(scale, par.at[1])
    sc = par[1, :]
    par[1, :] = np.float32(-0.5) / (sc * sc)
    plsc.subcore_barrier()

    ppw = pairs_total // NW
    extra = pairs_total % NW
    npair = ppw + jnp.where(wid < extra, 1, 0)
    pair_base = ppw * wid + jnp.minimum(wid, extra)
    nb = 2 * npair

    idx = (idx0, idx1)
    rows = (rows0, rows1)
    obuf = (obuf0, obuf1)
    sem_x = (sem_x0, sem_x1)
    sem_g = (sem_g0, sem_g1)
    sem_o = (sem_o0, sem_o1)

    def block_off(b):
        return (pair_base * 2 + b) * B

    def issue_idx(b, p):
        off = block_off(b)
        pltpu.async_copy(ei.at[0, pl.ds(off, B)], idx[p].at[pl.ds(0, B)],
                         sem_x[p])
        pltpu.async_copy(ei.at[1, pl.ds(off, B)], idx[p].at[pl.ds(B, B)],
                         sem_x[p])

    def wait_idx(b, p):
        pltpu.make_async_copy(ei.at[0, pl.ds(0, B)],
                              idx[p].at[pl.ds(0, B)], sem_x[p]).wait()
        pltpu.make_async_copy(ei.at[1, pl.ds(0, B)],
                              idx[p].at[pl.ds(B, B)], sem_x[p]).wait()

    def issue_gather(p):
        pltpu.async_copy(shared.at[idx[p]], rows[p], sem_g[p])

    def wait_gather(p):
        pltpu.make_async_copy(shared.at[idx[p]], rows[p], sem_g[p]).wait()

    def out_slice(b):
        return out.at[:, pl.ds((pair_base * 2 + b) * TC_PER_B, TC_PER_B)]

    def issue_out(b, p):
        pltpu.async_copy(obuf[p], out_slice(b), sem_o[p])

    def wait_out(p):
        pltpu.make_async_copy(obuf[p], out_slice(0), sem_o[p]).wait()

    @pl.when(nb >= 1)
    def _():
        issue_idx(0, 0)

    @pl.when(nb >= 2)
    def _():
        issue_idx(1, 1)

    @pl.when(nb >= 1)
    def _():
        wait_idx(0, 0)
        issue_gather(0)

    def pair(p, carry):
        for half in (0, 1):
            b = 2 * p + half
            q = 1 - half
            wait_gather(half)

            @pl.when(b + 2 < nb)
            def _():
                issue_idx(b + 2, half)

            @pl.when(b + 1 < nb)
            def _():
                wait_idx(b + 1, q)
                issue_gather(q)

            @pl.when(b >= 2)
            def _():
                wait_out(half)

            _compute_block(rows[half], obuf[half], par, B // L)
            issue_out(b, half)
        return carry

    lax.fori_loop(0, npair, pair, 0, unroll=False)

    @pl.when(nb >= 2)
    def _():
        wait_out(0)

    @pl.when(nb >= 1)
    def _():
        wait_out(1)

    @pl.when(wid < rem_tc)
    def _():
        tc0 = pairs_total * 2 * TC_PER_B + wid
        off = tc0 * 128

        def zero(k, carry):
            idx0[pl.ds(k * L, L)] = jnp.zeros((L,), jnp.int32)
            return carry

        lax.fori_loop(0, 2 * B // L, zero, 0, unroll=False)
        pltpu.sync_copy(ei.at[0, pl.ds(off, 128)], idx0.at[pl.ds(0, 128)])
        pltpu.sync_copy(ei.at[1, pl.ds(off, 128)], idx0.at[pl.ds(B, 128)])
        pltpu.async_copy(shared.at[idx0], rows0, sem_g0).wait()
        _compute_block(rows0, obuf0, par, 128 // L)
        pltpu.sync_copy(obuf0.at[:, pl.ds(0, 1)], out.at[:, pl.ds(tc0, 1)])


def kernel(pos, edge_index, shift, scale):
    n_nodes = pos.shape[0]
    n_edges = edge_index.shape[1]
    ei = edge_index.astype(jnp.int32)
    npad = -(-n_nodes // NS) * NS
    pos8 = jnp.pad(pos.astype(jnp.float32),
                   ((0, npad - n_nodes), (0, 8 - pos.shape[1])))

    e_pad = -(-n_edges // 128) * 128
    if e_pad != n_edges:
        ei = jnp.pad(ei, ((0, 0), (0, e_pad - n_edges)))
    ec = e_pad // 128
    pairs_total = ec // (2 * TC_PER_B)
    rem_tc = ec % (2 * TC_PER_B)
    assert rem_tc < NW

    mesh = plsc.VectorSubcoreMesh(core_axis_name="c", subcore_axis_name="s")
    f = pl.kernel(
        functools.partial(_body, npad, pairs_total, rem_tc),
        out_type=jax.ShapeDtypeStruct((2, ec, 8, 128), jnp.float32),
        mesh=mesh,
        scratch_types=[
            pltpu.VMEM_SHARED((npad, 8), jnp.float32),
            pltpu.VMEM((2 * B,), jnp.int32),
            pltpu.VMEM((2 * B,), jnp.int32),
            pltpu.VMEM((2 * B, 8), jnp.float32),
            pltpu.VMEM((2 * B, 8), jnp.float32),
            pltpu.VMEM((2, TC_PER_B, 8, 128), jnp.float32),
            pltpu.VMEM((2, TC_PER_B, 8, 128), jnp.float32),
            pltpu.VMEM((2, NUM_G), jnp.float32),
            pltpu.SemaphoreType.DMA,
            pltpu.SemaphoreType.DMA,
            pltpu.SemaphoreType.DMA,
            pltpu.SemaphoreType.DMA,
            pltpu.SemaphoreType.DMA,
            pltpu.SemaphoreType.DMA,
            pltpu.SemaphoreType.DMA,
        ],
        compiler_params=pltpu.CompilerParams(
            needs_layout_passes=False,
            use_tc_tiling_on_sc=False,
        ),
        name="gaussian_edge_embed_sc",
    )
    out = f(pos8, ei, shift.astype(jnp.float32), scale.astype(jnp.float32))
    res = out.transpose(1, 3, 0, 2).reshape(e_pad, NUM_G)
    if e_pad != n_edges:
        res = res[:n_edges]
    return res
